# baseline (speedup 1.0000x reference)
import functools

import jax
import jax.numpy as jnp
from jax import lax
from jax.experimental import pallas as pl
from jax.experimental.pallas import tpu as pltpu
from jax.experimental.pallas import tpu_sc as plsc

N = 10000
NPAD = 10240
B = 4096
E = 160000
EP = 163840
CHUNK = 128
CPW = EP // 32 // CHUNK
RPT = NPAD // 16


def make(mode, only_core, name, width=128):
    mesh = plsc.VectorSubcoreMesh(core_axis_name="c", subcore_axis_name="s")
    out_type = jax.ShapeDtypeStruct((2, NPAD, 128), jnp.float32)
    scratch = [
        pltpu.VMEM((CPW, CHUNK), jnp.int32),
        pltpu.VMEM((CPW, CHUNK), jnp.int32),
        pltpu.VMEM((CHUNK, width), jnp.float32),
        pltpu.VMEM((CHUNK, width), jnp.float32),
        pltpu.VMEM_SHARED((NPAD, 128), jnp.float32),
        pltpu.SemaphoreType.DMA,
        pltpu.SemaphoreType.DMA,
    ]

    @functools.partial(pl.kernel, out_type=out_type, mesh=mesh,
                       scratch_types=scratch, name=name)
    def k(src_hbm, dst_hbm, zeros_hbm, y_hbm, out, srcv, dstv, rows0, rows1,
          agg_sp, sem0, sem1):
        cid = lax.axis_index("c")
        sid = lax.axis_index("s")
        wid = sid * 2 + cid
        pltpu.sync_copy(src_hbm.at[pl.ds(wid * CPW, CPW)], srcv)
        pltpu.sync_copy(dst_hbm.at[pl.ds(wid * CPW, CPW)], dstv)
        pltpu.sync_copy(zeros_hbm, agg_sp.at[pl.ds(sid * RPT, RPT)])
        plsc.subcore_barrier()

        @pl.when((cid == only_core) | (only_core < 0))
        def _():
            if mode == "gather":
                pltpu.async_copy(y_hbm.at[srcv.at[0]], rows0, sem0)

                def body(jj, c):
                    j0, j1 = 2 * jj, 2 * jj + 1
                    pltpu.async_copy(y_hbm.at[srcv.at[j1]], rows1, sem1)
                    pltpu.make_async_copy(y_hbm.at[srcv.at[j0]], rows0, sem0).wait()

                    @pl.when(jj < CPW // 2 - 1)
                    def _():
                        pltpu.async_copy(y_hbm.at[srcv.at[j0 + 2]], rows0, sem0)
                    pltpu.make_async_copy(y_hbm.at[srcv.at[j1]], rows1, sem1).wait()
                    return c
                lax.fori_loop(0, CPW // 2, body, 0)
            elif mode == "scatter":
                def body(j, c):
                    pltpu.sync_copy(rows0.at[:, pl.ds(0, 128)], agg_sp.at[dstv.at[j]], add=True)
                    return c
                lax.fori_loop(0, CPW, body, 0)
            elif mode == "g1buf":
                def body(j, c):
                    pltpu.async_copy(y_hbm.at[srcv.at[j]], rows0, sem0).wait()
                    return c
                lax.fori_loop(0, CPW, body, 0)
            else:
                pltpu.async_copy(y_hbm.at[srcv.at[0]], rows0, sem0)

                def body(jj, c):
                    j0, j1 = 2 * jj, 2 * jj + 1
                    pltpu.async_copy(y_hbm.at[srcv.at[j1]], rows1, sem1)
                    pltpu.make_async_copy(y_hbm.at[srcv.at[j0]], rows0, sem0).wait()
                    pltpu.sync_copy(rows0, agg_sp.at[dstv.at[j0]], add=True)

                    @pl.when(jj < CPW // 2 - 1)
                    def _():
                        pltpu.async_copy(y_hbm.at[srcv.at[j0 + 2]], rows0, sem0)
                    pltpu.make_async_copy(y_hbm.at[srcv.at[j1]], rows1, sem1).wait()
                    pltpu.sync_copy(rows1, agg_sp.at[dstv.at[j1]], add=True)
                    return c
                lax.fori_loop(0, CPW // 2, body, 0)

        plsc.subcore_barrier()
        pltpu.sync_copy(agg_sp.at[pl.ds(sid * RPT, RPT)],
                        out.at[cid].at[pl.ds(sid * RPT, RPT)])

    return k


_tests = [
    ("gather", -1, "g_rand_a", 128, "rand", 10000),
    ("gather", -1, "g_rand_b", 128, "rand", 10000),
    ("gather", -1, "g_iota", 128, "iota", 10000),
    ("gather", -1, "g_hi", 128, "hi", 10000),
    ("gather", -1, "g_allsame", 128, "allsame", 10000),
]
_kernels = [(make(m, c, n, w), idx, tbl) for (m, c, n, w, idx, tbl) in _tests]


def kernel(user_indices, item_indices, edge_index, user_table, item_table,
           Wsrc0, Wdst0, b0, Wsrc1, Wdst1, b1, Wsrc2, Wdst2, b2,
           Wr1, br1, Wr2, br2):
    src = edge_index[0].astype(jnp.int32)
    dst = edge_index[1].astype(jnp.int32)
    pad = EP - E
    src_p = jnp.concatenate([src, jnp.zeros((pad,), jnp.int32)]
                            ).reshape(EP // CHUNK, CHUNK)
    dst_p = jnp.concatenate([dst, jnp.full((pad,), N, jnp.int32)]
                            ).reshape(EP // CHUNK, CHUNK)
    zeros128 = jnp.zeros((RPT, 128), jnp.float32)
    x = jnp.concatenate([user_table, item_table], axis=0)
    tables = {
        10000: x[:, :128] * 1.0,
        2048: x[:2048, :128] * 1.0,
    }
    srcs = {"rand": src_p,
            "iota": jnp.mod(jnp.arange(EP, dtype=jnp.int32), N).reshape(src_p.shape),
            "hi": 5000 + jnp.mod(src_p, 5000),
            "allsame": jnp.full(src_p.shape, 5000, jnp.int32)}
    y1024 = x * 1.0

    acc = jnp.zeros((B,), jnp.float32)
    for kf, idx, tbl in _kernels:
        out = kf(srcs[idx], dst_p, zeros128, tables[tbl])
        acc = acc + out[0, :B, 0]
    return acc


# speedup vs baseline: 1.6677x; 1.6677x over previous
import functools

import jax
import jax.numpy as jnp
from jax import lax
from jax.experimental import pallas as pl
from jax.experimental.pallas import tpu as pltpu
from jax.experimental.pallas import tpu_sc as plsc

NUM_USERS = 6000
NUM_ITEMS = 4000
N = NUM_USERS + NUM_ITEMS
NPAD = 10240
D = 256
B = 4096
E = 160000
EP = 172032
NW = 32
CHUNK = 112
CPW0 = 80
CPW1 = 16
NCH = EP // CHUNK
CPWC = NCH // NW
RPT = NPAD // 16
NB = 10
NBLK = N // NB


def _make_sc_agg(num_cb):
    mesh = plsc.VectorSubcoreMesh(core_axis_name="c", subcore_axis_name="s")
    out_type = [jax.ShapeDtypeStruct((2, NPAD, 128), jnp.float32)
                for _ in range(num_cb)]
    scratch = [
        pltpu.VMEM((CPW0, CHUNK), jnp.int32),
        pltpu.VMEM((CPW0, CHUNK), jnp.int32),
        pltpu.VMEM((CHUNK, 128), jnp.float32),
        pltpu.VMEM((CHUNK, 128), jnp.float32),
        pltpu.VMEM_SHARED((NPAD, 128), jnp.float32),
        pltpu.SemaphoreType.DMA,
        pltpu.SemaphoreType.DMA,
    ]

    @functools.partial(pl.kernel, out_type=out_type, mesh=mesh,
                       scratch_types=scratch)
    def k(src_hbm, dst_hbm, zeros_hbm, *rest):
        ys = rest[:num_cb]
        outs = rest[num_cb:2 * num_cb]
        srcv, dstv, rows0, rows1, agg_sp, sem0, sem1 = rest[2 * num_cb:]

        cid = lax.axis_index("c")
        sid = lax.axis_index("s")
        base = jnp.where(cid == 0, sid * CPW0, 16 * CPW0 + sid * CPW1)
        mycpw = jnp.where(cid == 0, CPW0, CPW1)

        pltpu.sync_copy(src_hbm.at[pl.ds(base, CPW0)], srcv)
        pltpu.sync_copy(dst_hbm.at[pl.ds(base, CPW0)], dstv)

        for cb in range(num_cb):
            pltpu.sync_copy(zeros_hbm, agg_sp.at[pl.ds(sid * RPT, RPT)])
            plsc.subcore_barrier()

            pltpu.async_copy(ys[cb].at[srcv.at[0]], rows0, sem0)

            def body(jj, carry):
                j0 = 2 * jj
                j1 = j0 + 1
                pltpu.async_copy(ys[cb].at[srcv.at[j1]], rows1, sem1)
                pltpu.make_async_copy(ys[cb].at[srcv.at[j0]], rows0,
                                      sem0).wait()
                pltpu.sync_copy(rows0, agg_sp.at[dstv.at[j0]], add=True)

                @pl.when(jj < mycpw // 2 - 1)
                def _():
                    pltpu.async_copy(ys[cb].at[srcv.at[j0 + 2]], rows0, sem0)

                pltpu.make_async_copy(ys[cb].at[srcv.at[j1]], rows1,
                                      sem1).wait()
                pltpu.sync_copy(rows1, agg_sp.at[dstv.at[j1]], add=True)
                return carry

            lax.fori_loop(0, mycpw // 2, body, 0)
            plsc.subcore_barrier()

            pltpu.sync_copy(agg_sp.at[pl.ds(sid * RPT, RPT)],
                            outs[cb].at[cid].at[pl.ds(sid * RPT, RPT)])

    return k


def _make_sc_count():
    mesh = plsc.VectorSubcoreMesh(core_axis_name="c", subcore_axis_name="s")
    out_type = jax.ShapeDtypeStruct((2, NPAD, 128), jnp.float32)
    scratch = [
        pltpu.VMEM((CPWC, CHUNK), jnp.int32),
        pltpu.VMEM((CHUNK, 128), jnp.float32),
        pltpu.VMEM_SHARED((NPAD, 128), jnp.float32),
    ]

    @functools.partial(pl.kernel, out_type=out_type, mesh=mesh,
                       scratch_types=scratch)
    def k(dst_hbm, zeros16_hbm, ones16_hbm, cnt_out, dstv, onesv, cnt_sp):
        cid = lax.axis_index("c")
        sid = lax.axis_index("s")
        wid = sid * 2 + cid

        pltpu.sync_copy(dst_hbm.at[pl.ds(wid * CPWC, CPWC)], dstv)
        pltpu.sync_copy(ones16_hbm, onesv)
        pltpu.sync_copy(zeros16_hbm, cnt_sp.at[pl.ds(sid * RPT, RPT)])
        plsc.subcore_barrier()

        def body(j, carry):
            pltpu.sync_copy(onesv, cnt_sp.at[dstv.at[j]], add=True)
            return carry

        lax.fori_loop(0, CPWC, body, 0)
        plsc.subcore_barrier()
        pltpu.sync_copy(cnt_sp.at[pl.ds(sid * RPT, RPT)],
                        cnt_out.at[cid].at[pl.ds(sid * RPT, RPT)])

    return k


_sc_agg_l0 = _make_sc_agg(4)
_sc_agg_sm = _make_sc_agg(2)
_sc_count = _make_sc_count()


def _sc_pair_gather():
    mesh = plsc.VectorSubcoreMesh(core_axis_name="c", subcore_axis_name="s")
    rpw = B // NW
    out_type = [jax.ShapeDtypeStruct((B, D), jnp.float32),
                jax.ShapeDtypeStruct((B, D), jnp.float32)]
    scratch = [
        pltpu.VMEM((rpw,), jnp.int32),
        pltpu.VMEM((rpw, D), jnp.float32),
        pltpu.SemaphoreType.DMA,
    ]

    @functools.partial(pl.kernel, out_type=out_type, mesh=mesh,
                       scratch_types=scratch)
    def k(h_hbm, uidx_hbm, vidx_hbm, u_out, v_out, idxv, rows, sem):
        cid = lax.axis_index("c")
        sid = lax.axis_index("s")
        wid = sid * 2 + cid
        base = wid * rpw
        pltpu.sync_copy(uidx_hbm.at[pl.ds(base, rpw)], idxv)
        pltpu.async_copy(h_hbm.at[idxv], rows, sem).wait()
        pltpu.sync_copy(rows, u_out.at[pl.ds(base, rpw)])
        pltpu.sync_copy(vidx_hbm.at[pl.ds(base, rpw)], idxv)
        pltpu.async_copy(h_hbm.at[idxv], rows, sem).wait()
        pltpu.sync_copy(rows, v_out.at[pl.ds(base, rpw)])

    return k


_sc_gather_uv = _sc_pair_gather()


def _mm_cols(x, w):
    din, dout = w.shape
    num_cb = dout // 128

    def body(x_ref, w_ref, *out_refs):
        acc = jnp.dot(x_ref[...], w_ref[...],
                      preferred_element_type=jnp.float32)
        for cb in range(num_cb):
            out_refs[cb][...] = acc[:, cb * 128:(cb + 1) * 128]

    return pl.pallas_call(
        body,
        grid=(NB,),
        in_specs=[pl.BlockSpec((NBLK, din), lambda i: (i, 0)),
                  pl.BlockSpec((din, dout), lambda i: (0, 0))],
        out_specs=[pl.BlockSpec((NBLK, 128), lambda i: (i, 0))] * num_cb,
        out_shape=[jax.ShapeDtypeStruct((N, 128), jnp.float32)] * num_cb,
    )(x, w)


def _combine(parts, cnt_part, x_prev, wd, b, res=None):
    din, dout = wd.shape
    num_cb = len(parts)
    with_res = res is not None

    in_specs = ([pl.BlockSpec((2, NBLK, 128), lambda i: (0, i, 0))
                 ] * num_cb +
                [pl.BlockSpec((2, NBLK, 128), lambda i: (0, i, 0)),
                 pl.BlockSpec((NBLK, din), lambda i: (i, 0)),
                 pl.BlockSpec((din, dout), lambda i: (0, 0)),
                 pl.BlockSpec((1, dout), lambda i: (0, 0))])
    args = list(parts) + [cnt_part, x_prev, wd, b.reshape(1, dout)]
    if with_res:
        in_specs.append(pl.BlockSpec((NBLK, dout), lambda i: (i, 0)))
        args.append(res)

    def body2(*refs):
        part_refs = refs[:num_cb]
        rest = refs[num_cb:]
        ps = [r[...] for r in part_refs]
        agg = jnp.concatenate([p[0] + p[1] for p in ps], axis=-1)
        cnt_ref, x_ref, wd_ref, b_ref = rest[:4]
        pos = 4
        if with_res:
            res_ref = rest[pos]
            pos += 1
        out_ref = rest[pos]
        c = cnt_ref[...]
        cnt = c[0, :, 0:1] + c[1, :, 0:1]
        agg = agg / jnp.maximum(cnt, 1.0)
        d = jnp.dot(x_ref[...], wd_ref[...],
                    preferred_element_type=jnp.float32)
        o = agg + d
        nrm = jnp.sqrt(jnp.sum(o * o, axis=-1, keepdims=True))
        o = o / jnp.maximum(nrm, 1e-12) + b_ref[...]
        if with_res:
            o = o + res_ref[...]
        out_ref[...] = o

    return pl.pallas_call(
        body2,
        grid=(NB,),
        in_specs=in_specs,
        out_specs=pl.BlockSpec((NBLK, dout), lambda i: (i, 0)),
        out_shape=jax.ShapeDtypeStruct((N, dout), jnp.float32),
    )(*args)


def _mlp(u, v, w1u, w1v, b1, w2row, b2):
    nb = 1024

    def body(u_ref, v_ref, w1u_ref, w1v_ref, b1_ref, w2_ref, b2_ref, out_ref):
        z = (jnp.dot(u_ref[...], w1u_ref[...],
                     preferred_element_type=jnp.float32) +
             jnp.dot(v_ref[...], w1v_ref[...],
                     preferred_element_type=jnp.float32) + b1_ref[...])
        z = jnp.maximum(z, 0.0)
        out_ref[...] = (jnp.sum(z * w2_ref[...], axis=-1, keepdims=True)
                        + b2_ref[...])

    return pl.pallas_call(
        body,
        grid=(B // nb,),
        in_specs=[pl.BlockSpec((nb, D), lambda i: (i, 0)),
                  pl.BlockSpec((nb, D), lambda i: (i, 0)),
                  pl.BlockSpec((D, D), lambda i: (0, 0)),
                  pl.BlockSpec((D, D), lambda i: (0, 0)),
                  pl.BlockSpec((1, D), lambda i: (0, 0)),
                  pl.BlockSpec((1, D), lambda i: (0, 0)),
                  pl.BlockSpec((1, 1), lambda i: (0, 0))],
        out_specs=pl.BlockSpec((nb, 1), lambda i: (i, 0)),
        out_shape=jax.ShapeDtypeStruct((B, 1), jnp.float32),
    )(u, v, w1u, w1v, b1, w2row, b2)


def kernel(user_indices, item_indices, edge_index, user_table, item_table,
           Wsrc0, Wdst0, b0, Wsrc1, Wdst1, b1, Wsrc2, Wdst2, b2,
           Wr1, br1, Wr2, br2):
    x = jnp.concatenate([user_table, item_table], axis=0)

    src = edge_index[0].astype(jnp.int32)
    dst = edge_index[1].astype(jnp.int32)
    pad = EP - E
    src_p = jnp.concatenate([src, jnp.zeros((pad,), jnp.int32)]
                            ).reshape(EP // CHUNK, CHUNK)
    dst_p = jnp.concatenate([dst, jnp.full((pad,), N, jnp.int32)]
                            ).reshape(EP // CHUNK, CHUNK)

    zeros128 = jnp.zeros((RPT, 128), jnp.float32)
    ones128 = jnp.ones((CHUNK, 128), jnp.float32)

    cnt_part = _sc_count(dst_p, zeros128, ones128)

    y0 = _mm_cols(x, Wsrc0)
    parts0 = _sc_agg_l0(src_p, dst_p, zeros128, *y0)
    h0 = _combine(parts0, cnt_part, x, Wdst0, b0)

    y1 = _mm_cols(h0, Wsrc1)
    parts1 = _sc_agg_sm(src_p, dst_p, zeros128, *y1)
    h1 = _combine(parts1, cnt_part, h0, Wdst1, b1)

    y2 = _mm_cols(h1, Wsrc2)
    parts2 = _sc_agg_sm(src_p, dst_p, zeros128, *y2)
    h = _combine(parts2, cnt_part, h1, Wdst2, b2, res=h1)

    uidx = user_indices.astype(jnp.int32)
    vidx = item_indices.astype(jnp.int32) + NUM_USERS
    u, v = _sc_gather_uv(h, uidx, vidx)
    r = _mlp(u, v, Wr1[:D], Wr1[D:], br1.reshape(1, D),
             Wr2[:, 0].reshape(1, D), br2.reshape(1, 1))
    return r[:, 0]


# speedup vs baseline: 10.1898x; 6.1100x over previous
import functools

import jax
import jax.numpy as jnp
from jax import lax
from jax.experimental import pallas as pl
from jax.experimental.pallas import tpu as pltpu
from jax.experimental.pallas import tpu_sc as plsc

NUM_USERS = 6000
NUM_ITEMS = 4000
N = NUM_USERS + NUM_ITEMS
NPAD = 10240
D = 256
B = 4096
E = 160000
EP = 163840
NW = 32
CHUNK = 128
CPW = EP // NW // CHUNK
RPT = NPAD // 16
NB = 10
NBLK = N // NB


def _make_sc_agg(num_cb):
    mesh = plsc.VectorSubcoreMesh(core_axis_name="c", subcore_axis_name="s")
    out_type = [jax.ShapeDtypeStruct((2, NPAD, 128), jnp.float32)
                for _ in range(num_cb)]
    scratch = [
        pltpu.VMEM((CPW, CHUNK), jnp.int32),
        pltpu.VMEM((CPW, CHUNK), jnp.int32),
        pltpu.VMEM((CHUNK, 128), jnp.float32),
        pltpu.VMEM((CHUNK, 128), jnp.float32),
        pltpu.VMEM_SHARED((NPAD, 128), jnp.float32),
        pltpu.SemaphoreType.DMA,
        pltpu.SemaphoreType.DMA,
    ]

    @functools.partial(pl.kernel, out_type=out_type, mesh=mesh,
                       scratch_types=scratch)
    def k(src_hbm, dst_hbm, zeros_hbm, *rest):
        ys = rest[:num_cb]
        outs = rest[num_cb:2 * num_cb]
        srcv, dstv, rows0, rows1, agg_sp, sem0, sem1 = rest[2 * num_cb:]

        cid = lax.axis_index("c")
        sid = lax.axis_index("s")
        wid = sid * 2 + cid

        pltpu.sync_copy(src_hbm.at[pl.ds(wid * CPW, CPW)], srcv)
        pltpu.sync_copy(dst_hbm.at[pl.ds(wid * CPW, CPW)], dstv)

        for cb in range(num_cb):
            pltpu.sync_copy(zeros_hbm, agg_sp.at[pl.ds(sid * RPT, RPT)])
            plsc.subcore_barrier()

            pltpu.async_copy(ys[cb].at[srcv.at[0]], rows0, sem0)

            def body(jj, carry):
                j0 = 2 * jj
                j1 = j0 + 1
                pltpu.async_copy(ys[cb].at[srcv.at[j1]], rows1, sem1)
                pltpu.make_async_copy(ys[cb].at[srcv.at[j0]], rows0,
                                      sem0).wait()
                pltpu.sync_copy(rows0, agg_sp.at[dstv.at[j0]], add=True)

                @pl.when(jj < CPW // 2 - 1)
                def _():
                    pltpu.async_copy(ys[cb].at[srcv.at[j0 + 2]], rows0, sem0)

                pltpu.make_async_copy(ys[cb].at[srcv.at[j1]], rows1,
                                      sem1).wait()
                pltpu.sync_copy(rows1, agg_sp.at[dstv.at[j1]], add=True)
                return carry

            lax.fori_loop(0, CPW // 2, body, 0)
            plsc.subcore_barrier()

            pltpu.sync_copy(agg_sp.at[pl.ds(sid * RPT, RPT)],
                            outs[cb].at[cid].at[pl.ds(sid * RPT, RPT)])

    return k


def _make_sc_count():
    mesh = plsc.VectorSubcoreMesh(core_axis_name="c", subcore_axis_name="s")
    out_type = jax.ShapeDtypeStruct((2, NPAD, 128), jnp.float32)
    scratch = [
        pltpu.VMEM((CPW, CHUNK), jnp.int32),
        pltpu.VMEM((CHUNK, 128), jnp.float32),
        pltpu.VMEM_SHARED((NPAD, 128), jnp.float32),
    ]

    @functools.partial(pl.kernel, out_type=out_type, mesh=mesh,
                       scratch_types=scratch)
    def k(dst_hbm, zeros16_hbm, ones16_hbm, cnt_out, dstv, onesv, cnt_sp):
        cid = lax.axis_index("c")
        sid = lax.axis_index("s")
        wid = sid * 2 + cid

        pltpu.sync_copy(dst_hbm.at[pl.ds(wid * CPW, CPW)], dstv)
        pltpu.sync_copy(ones16_hbm, onesv)
        pltpu.sync_copy(zeros16_hbm, cnt_sp.at[pl.ds(sid * RPT, RPT)])
        plsc.subcore_barrier()

        def body(j, carry):
            pltpu.sync_copy(onesv, cnt_sp.at[dstv.at[j]], add=True)
            return carry

        lax.fori_loop(0, CPW, body, 0)
        plsc.subcore_barrier()
        pltpu.sync_copy(cnt_sp.at[pl.ds(sid * RPT, RPT)],
                        cnt_out.at[cid].at[pl.ds(sid * RPT, RPT)])

    return k


_sc_agg_l0 = _make_sc_agg(4)
_sc_agg_sm = _make_sc_agg(2)
_sc_count = _make_sc_count()


def _sc_pair_gather():
    mesh = plsc.VectorSubcoreMesh(core_axis_name="c", subcore_axis_name="s")
    rpw = B // NW
    out_type = [jax.ShapeDtypeStruct((B, D), jnp.float32),
                jax.ShapeDtypeStruct((B, D), jnp.float32)]
    scratch = [
        pltpu.VMEM((rpw,), jnp.int32),
        pltpu.VMEM((rpw, D), jnp.float32),
        pltpu.SemaphoreType.DMA,
    ]

    @functools.partial(pl.kernel, out_type=out_type, mesh=mesh,
                       scratch_types=scratch)
    def k(h_hbm, uidx_hbm, vidx_hbm, u_out, v_out, idxv, rows, sem):
        cid = lax.axis_index("c")
        sid = lax.axis_index("s")
        wid = sid * 2 + cid
        base = wid * rpw
        pltpu.sync_copy(uidx_hbm.at[pl.ds(base, rpw)], idxv)
        pltpu.async_copy(h_hbm.at[idxv], rows, sem).wait()
        pltpu.sync_copy(rows, u_out.at[pl.ds(base, rpw)])
        pltpu.sync_copy(vidx_hbm.at[pl.ds(base, rpw)], idxv)
        pltpu.async_copy(h_hbm.at[idxv], rows, sem).wait()
        pltpu.sync_copy(rows, v_out.at[pl.ds(base, rpw)])

    return k


_sc_gather_uv = _sc_pair_gather()


def _mm_cols(x, w):
    din, dout = w.shape
    num_cb = dout // 128

    def body(x_ref, w_ref, *out_refs):
        acc = jnp.dot(x_ref[...], w_ref[...],
                      preferred_element_type=jnp.float32)
        for cb in range(num_cb):
            out_refs[cb][...] = acc[:, cb * 128:(cb + 1) * 128]

    return pl.pallas_call(
        body,
        grid=(NB,),
        in_specs=[pl.BlockSpec((NBLK, din), lambda i: (i, 0)),
                  pl.BlockSpec((din, dout), lambda i: (0, 0))],
        out_specs=[pl.BlockSpec((NBLK, 128), lambda i: (i, 0))] * num_cb,
        out_shape=[jax.ShapeDtypeStruct((N, 128), jnp.float32)] * num_cb,
    )(x, w)


def _combine(parts, cnt_part, x_prev, wd, b, res=None):
    din, dout = wd.shape
    num_cb = len(parts)
    with_res = res is not None

    in_specs = ([pl.BlockSpec((2, NBLK, 128), lambda i: (0, i, 0))
                 ] * num_cb +
                [pl.BlockSpec((2, NBLK, 128), lambda i: (0, i, 0)),
                 pl.BlockSpec((NBLK, din), lambda i: (i, 0)),
                 pl.BlockSpec((din, dout), lambda i: (0, 0)),
                 pl.BlockSpec((1, dout), lambda i: (0, 0))])
    args = list(parts) + [cnt_part, x_prev, wd, b.reshape(1, dout)]
    if with_res:
        in_specs.append(pl.BlockSpec((NBLK, dout), lambda i: (i, 0)))
        args.append(res)

    def body2(*refs):
        part_refs = refs[:num_cb]
        rest = refs[num_cb:]
        ps = [r[...] for r in part_refs]
        agg = jnp.concatenate([p[0] + p[1] for p in ps], axis=-1)
        cnt_ref, x_ref, wd_ref, b_ref = rest[:4]
        pos = 4
        if with_res:
            res_ref = rest[pos]
            pos += 1
        out_ref = rest[pos]
        c = cnt_ref[...]
        cnt = c[0, :, 0:1] + c[1, :, 0:1]
        agg = agg / jnp.maximum(cnt, 1.0)
        d = jnp.dot(x_ref[...], wd_ref[...],
                    preferred_element_type=jnp.float32)
        o = agg + d
        nrm = jnp.sqrt(jnp.sum(o * o, axis=-1, keepdims=True))
        o = o / jnp.maximum(nrm, 1e-12) + b_ref[...]
        if with_res:
            o = o + res_ref[...]
        out_ref[...] = o

    return pl.pallas_call(
        body2,
        grid=(NB,),
        in_specs=in_specs,
        out_specs=pl.BlockSpec((NBLK, dout), lambda i: (i, 0)),
        out_shape=jax.ShapeDtypeStruct((N, dout), jnp.float32),
    )(*args)


def _mlp(u, v, w1u, w1v, b1, w2row, b2):
    nb = 1024

    def body(u_ref, v_ref, w1u_ref, w1v_ref, b1_ref, w2_ref, b2_ref, out_ref):
        z = (jnp.dot(u_ref[...], w1u_ref[...],
                     preferred_element_type=jnp.float32) +
             jnp.dot(v_ref[...], w1v_ref[...],
                     preferred_element_type=jnp.float32) + b1_ref[...])
        z = jnp.maximum(z, 0.0)
        out_ref[...] = (jnp.sum(z * w2_ref[...], axis=-1, keepdims=True)
                        + b2_ref[...])

    return pl.pallas_call(
        body,
        grid=(B // nb,),
        in_specs=[pl.BlockSpec((nb, D), lambda i: (i, 0)),
                  pl.BlockSpec((nb, D), lambda i: (i, 0)),
                  pl.BlockSpec((D, D), lambda i: (0, 0)),
                  pl.BlockSpec((D, D), lambda i: (0, 0)),
                  pl.BlockSpec((1, D), lambda i: (0, 0)),
                  pl.BlockSpec((1, D), lambda i: (0, 0)),
                  pl.BlockSpec((1, 1), lambda i: (0, 0))],
        out_specs=pl.BlockSpec((nb, 1), lambda i: (i, 0)),
        out_shape=jax.ShapeDtypeStruct((B, 1), jnp.float32),
    )(u, v, w1u, w1v, b1, w2row, b2)


def kernel(user_indices, item_indices, edge_index, user_table, item_table,
           Wsrc0, Wdst0, b0, Wsrc1, Wdst1, b1, Wsrc2, Wdst2, b2,
           Wr1, br1, Wr2, br2):
    x = jnp.concatenate([user_table, item_table], axis=0)

    src = edge_index[0].astype(jnp.int32)
    dst = edge_index[1].astype(jnp.int32)
    pad = EP - E
    src_p = jnp.concatenate([src, jnp.arange(pad, dtype=jnp.int32) % N]
                            ).reshape(EP // CHUNK, CHUNK)
    dst_p = jnp.concatenate([dst, jnp.full((pad,), N, jnp.int32)]
                            ).reshape(EP // CHUNK, CHUNK)

    zeros128 = jnp.zeros((RPT, 128), jnp.float32)
    ones128 = jnp.ones((CHUNK, 128), jnp.float32)

    cnt_part = _sc_count(dst_p, zeros128, ones128)

    y0 = _mm_cols(x, Wsrc0)
    parts0 = _sc_agg_l0(src_p, dst_p, zeros128, *y0)
    h0 = _combine(parts0, cnt_part, x, Wdst0, b0)

    y1 = _mm_cols(h0, Wsrc1)
    parts1 = _sc_agg_sm(src_p, dst_p, zeros128, *y1)
    h1 = _combine(parts1, cnt_part, h0, Wdst1, b1)

    y2 = _mm_cols(h1, Wsrc2)
    parts2 = _sc_agg_sm(src_p, dst_p, zeros128, *y2)
    h = _combine(parts2, cnt_part, h1, Wdst2, b2, res=h1)

    uidx = user_indices.astype(jnp.int32)
    vidx = item_indices.astype(jnp.int32) + NUM_USERS
    u, v = _sc_gather_uv(h, uidx, vidx)
    r = _mlp(u, v, Wr1[:D], Wr1[D:], br1.reshape(1, D),
             Wr2[:, 0].reshape(1, D), br2.reshape(1, 1))
    return r[:, 0]
